# baseline (speedup 1.0000x reference)
import functools

import jax
import jax.numpy as jnp
from jax import lax
from jax.experimental import pallas as pl
from jax.experimental.pallas import tpu as pltpu
from jax.experimental.pallas import tpu_sc as plsc

_NUM_EMB = 1000000
_D = 32
_B = 16384
_F = 26
_TOT = _B * _F
_NQ = _NUM_EMB // 4

_NC = 2
_NS = 16
_NW = _NC * _NS

_mesh = plsc.VectorSubcoreMesh(core_axis_name="c", subcore_axis_name="s")
_tc_params = pltpu.CompilerParams(
    use_tc_tiling_on_sc=True, needs_layout_passes=False)

_NBLK_FULL = _NUM_EMB // 128
_BPT = 245


@functools.partial(
    pl.kernel,
    mesh=_mesh,
    compiler_params=_tc_params,
    out_type=jax.ShapeDtypeStruct((_NQ, 128), jnp.float32),
    scratch_types=[
        pltpu.VMEM((2, 32, 128), jnp.float32),
        pltpu.VMEM((2, 32, 128), jnp.float32),
        pltpu.SemaphoreType.DMA,
        pltpu.SemaphoreType.DMA,
        pltpu.SemaphoreType.DMA,
        pltpu.SemaphoreType.DMA,
    ],
)
def _conv(tT_hbm, tail_hbm, xp_hbm, s_v, o_v, si0, si1, so0, so1):
    wid = lax.axis_index("s") * _NC + lax.axis_index("c")
    base = wid * _BPT
    isems = (si0, si1)
    osems = (so0, so1)
    iota = lax.iota(jnp.int32, 16)
    iota_d4 = iota // 4
    iota_m4_32 = (iota % 4) * 32

    def start_in(g, b):
        gblk = base + g

        @pl.when(gblk < _NBLK_FULL)
        def _():
            r0 = pl.multiple_of(gblk * 128, 128)
            pltpu.async_copy(
                tT_hbm.at[:, pl.ds(r0, 128)], s_v.at[b], isems[b])

    def wait_in(g, b):
        gblk = base + g

        @pl.when(gblk < _NBLK_FULL)
        def _():
            pltpu.make_async_copy(
                tT_hbm.at[:, pl.ds(0, 128)], s_v.at[b], isems[b]).wait()

    def shuffle(g, b):
        gblk = base + g

        @pl.when(gblk < _NBLK_FULL)
        def _():
            for rl0 in range(0, 128, 16):
                rowv = (rl0 // 4) + iota_d4
                for c in range(32):
                    v = s_v[b, c, pl.ds(rl0, 16)]
                    plsc.store_scatter(
                        o_v.at[b], [rowv, iota_m4_32 + c], v)

    def start_out(g, b):
        gblk = base + g

        @pl.when(gblk < _NBLK_FULL)
        def _():
            q0 = pl.multiple_of(gblk * 32, 32)
            pltpu.async_copy(o_v.at[b], xp_hbm.at[pl.ds(q0, 32)], osems[b])

    def wait_out(g, b):
        gblk = base + g

        @pl.when(gblk < _NBLK_FULL)
        def _():
            pltpu.make_async_copy(
                o_v.at[b], xp_hbm.at[pl.ds(0, 32)], osems[b]).wait()

    start_in(0, 0)
    start_in(1, 1)

    def body(k, carry):
        for b in range(2):
            g = 2 * k + b
            wait_in(g, b)

            @pl.when(g >= 2)
            def _():
                wait_out(g - 2, b)

            shuffle(g, b)
            start_out(g, b)

            @pl.when(g + 2 < _BPT)
            def _():
                start_in(g + 2, b)
        return carry

    lax.fori_loop(0, _BPT // 2, body, 0)
    g_last = _BPT - 1
    wait_in(g_last, 0)
    wait_out(g_last - 2, 0)
    shuffle(g_last, 0)
    start_out(g_last, 0)
    wait_out(g_last - 1, 1)
    wait_out(g_last, 0)

    @pl.when(wid == _NW - 1)
    def _():
        pltpu.sync_copy(tail_hbm, s_v.at[0, pl.ds(0, 16)])
        pltpu.sync_copy(s_v.at[0, pl.ds(0, 16)],
                        xp_hbm.at[pl.ds(_NBLK_FULL * 32, 16)])


_HB = 8
_HR = _F * _HB
_NH = 512 // _HB

_sc_params = pltpu.CompilerParams(
    use_tc_tiling_on_sc=False, needs_layout_passes=False)


@functools.partial(
    pl.kernel,
    mesh=_mesh,
    compiler_params=_sc_params,
    out_type=jax.ShapeDtypeStruct((_F, 4, 128, 8, 128), jnp.float32),
    scratch_types=[
        pltpu.VMEM((13312,), jnp.int32),
        pltpu.VMEM((2, _HR, 128), jnp.float32),
        pltpu.VMEM((2, _HR), jnp.int32),
        pltpu.VMEM((2, _HR), jnp.int32),
        pltpu.VMEM((_F, 4, 1, 8, 16), jnp.float32),
        pltpu.SemaphoreType.DMA,
        pltpu.SemaphoreType.DMA,
        pltpu.SemaphoreType.DMA,
        pltpu.SemaphoreType.DMA,
    ],
)
def _gath(xp_hbm, idx_hbm, out5_hbm, idx_v, g_v, q_v, s_v, mo_v,
          gs0, gs1, osem, isem):
    cid = lax.axis_index("c")
    sid = lax.axis_index("s")
    wid = sid * _NC + cid
    gsems = (gs0, gs1)
    iota = lax.iota(jnp.int32, 16)
    zerov = iota * 0

    ipos = pl.multiple_of(wid * 13312, 1024)
    pltpu.async_copy(idx_hbm.at[pl.ds(ipos, 13312)], idx_v, isem).wait()

    def prep(h, b):
        off = h * _HR
        for j in range(_HR // 16):
            x = idx_v[pl.ds(off + j * 16, 16)]
            q_v[b, pl.ds(j * 16, 16)] = lax.shift_right_logical(x, 2)
            s_v[b, pl.ds(j * 16, 16)] = lax.shift_left(x & 3, 5)

    def start_g(b):
        pltpu.async_copy(xp_hbm.at[q_v.at[b]], g_v.at[b], gsems[b])

    def wait_g(b):
        pltpu.make_async_copy(
            xp_hbm.at[pl.ds(0, _HR)], g_v.at[b], gsems[b]).wait()

    def shuffle(b, blbase):
        def grp(j, carry):
            iv = j * 16 + iota
            f_v = iv % _F
            bl_v = iv // _F + blbase
            sv = plsc.load_gather(s_v.at[b], [iv])
            for c in range(32):
                tcv = jnp.full((16,), c // 8, jnp.int32)
                c8v = jnp.full((16,), c % 8, jnp.int32)
                v = plsc.load_gather(g_v.at[b], [iv, sv + c])
                plsc.store_scatter(mo_v, [f_v, tcv, zerov, c8v, bl_v], v)
            return carry

        lax.fori_loop(0, _HR // 16, grp, 0)

    def start_out(sc):
        b0 = wid * 512 + sc * 16
        tr = b0 // 128
        bo = pl.multiple_of(b0 % 128, 16)
        pltpu.async_copy(
            mo_v, out5_hbm.at[:, :, pl.ds(tr, 1), :, pl.ds(bo, 16)], osem)

    def wait_out():
        pltpu.make_async_copy(
            mo_v, out5_hbm.at[:, :, pl.ds(0, 1), :, pl.ds(0, 16)],
            osem).wait()

    prep(0, 0)
    start_g(0)
    prep(1, 1)
    start_g(1)

    def body(k, carry):
        for b in range(2):
            h = 2 * k + b
            wait_g(b)

            @pl.when((b == 0) & (k >= 1))
            def _():
                wait_out()

            shuffle(b, b * _HB)

            @pl.when(h + 2 < _NH)
            def _():
                prep(h + 2, b)
                start_g(b)

        start_out(k)
        return carry

    lax.fori_loop(0, _NH // 2, body, 0)
    wait_out()


def kernel(indices, table):
    idx = indices.astype(jnp.int32).reshape(_TOT)
    tail = lax.slice(table, (_NBLK_FULL * 128, 0), (_NUM_EMB, _D))
    tail16 = tail.reshape(16, 128)
    xp = _conv(table.T, tail16)
    out5 = _gath(xp, idx)
    return jnp.transpose(out5, (2, 4, 0, 1, 3)).reshape(_B, _F, _D)


# speedup vs baseline: 1.1739x; 1.1739x over previous
import functools

import jax
import jax.numpy as jnp
from jax import lax
from jax.experimental import pallas as pl
from jax.experimental.pallas import tpu as pltpu
from jax.experimental.pallas import tpu_sc as plsc

_NUM_EMB = 1000000
_D = 32
_B = 16384
_F = 26
_TOT = _B * _F
_NQ = _NUM_EMB // 4

_NC = 2
_NS = 16
_NW = _NC * _NS

_mesh = plsc.VectorSubcoreMesh(core_axis_name="c", subcore_axis_name="s")
_tc_params = pltpu.CompilerParams(
    use_tc_tiling_on_sc=True, needs_layout_passes=False)

_NBLK_FULL = _NUM_EMB // 128
_BPT = 245


@functools.partial(
    pl.kernel,
    mesh=_mesh,
    compiler_params=_tc_params,
    out_type=jax.ShapeDtypeStruct((_NQ, 128), jnp.float32),
    scratch_types=[
        pltpu.VMEM((2, 32, 128), jnp.float32),
        pltpu.VMEM((2, 32, 128), jnp.float32),
        pltpu.SemaphoreType.DMA,
        pltpu.SemaphoreType.DMA,
        pltpu.SemaphoreType.DMA,
        pltpu.SemaphoreType.DMA,
    ],
)
def _conv(tT_hbm, tail_hbm, xp_hbm, s_v, o_v, si0, si1, so0, so1):
    wid = lax.axis_index("s") * _NC + lax.axis_index("c")
    base = wid * _BPT
    isems = (si0, si1)
    osems = (so0, so1)
    iota = lax.iota(jnp.int32, 16)
    iota_d4 = iota // 4
    iota_m4_32 = (iota % 4) * 32

    def start_in(g, b):
        gblk = base + g

        @pl.when(gblk < _NBLK_FULL)
        def _():
            r0 = pl.multiple_of(gblk * 128, 128)
            pltpu.async_copy(
                tT_hbm.at[:, pl.ds(r0, 128)], s_v.at[b], isems[b])

    def wait_in(g, b):
        gblk = base + g

        @pl.when(gblk < _NBLK_FULL)
        def _():
            pltpu.make_async_copy(
                tT_hbm.at[:, pl.ds(0, 128)], s_v.at[b], isems[b]).wait()

    def shuffle(g, b):
        gblk = base + g

        @pl.when(gblk < _NBLK_FULL)
        def _():
            for rl0 in range(0, 128, 16):
                rowv = (rl0 // 4) + iota_d4
                for c in range(32):
                    v = s_v[b, c, pl.ds(rl0, 16)]
                    plsc.store_scatter(
                        o_v.at[b], [rowv, iota_m4_32 + c], v)

    def start_out(g, b):
        gblk = base + g

        @pl.when(gblk < _NBLK_FULL)
        def _():
            q0 = pl.multiple_of(gblk * 32, 32)
            pltpu.async_copy(o_v.at[b], xp_hbm.at[pl.ds(q0, 32)], osems[b])

    def wait_out(g, b):
        gblk = base + g

        @pl.when(gblk < _NBLK_FULL)
        def _():
            pltpu.make_async_copy(
                o_v.at[b], xp_hbm.at[pl.ds(0, 32)], osems[b]).wait()

    start_in(0, 0)
    start_in(1, 1)

    def body(k, carry):
        for b in range(2):
            g = 2 * k + b
            wait_in(g, b)

            @pl.when(g >= 2)
            def _():
                wait_out(g - 2, b)

            shuffle(g, b)
            start_out(g, b)

            @pl.when(g + 2 < _BPT)
            def _():
                start_in(g + 2, b)
        return carry

    lax.fori_loop(0, _BPT // 2, body, 0)
    g_last = _BPT - 1
    wait_in(g_last, 0)
    wait_out(g_last - 2, 0)
    shuffle(g_last, 0)
    start_out(g_last, 0)
    wait_out(g_last - 1, 1)
    wait_out(g_last, 0)

    @pl.when(wid == _NW - 1)
    def _():
        pltpu.sync_copy(tail_hbm, s_v.at[0, pl.ds(0, 16)])
        pltpu.sync_copy(s_v.at[0, pl.ds(0, 16)],
                        xp_hbm.at[pl.ds(_NBLK_FULL * 32, 16)])


_HB = 8
_HR = _F * _HB
_NH = 512 // _HB

_sc_params = pltpu.CompilerParams(
    use_tc_tiling_on_sc=False, needs_layout_passes=False)


@functools.partial(
    pl.kernel,
    mesh=_mesh,
    compiler_params=_sc_params,
    out_type=jax.ShapeDtypeStruct((_F, 4, 128, 8, 128), jnp.float32),
    scratch_types=[
        pltpu.VMEM((13312,), jnp.int32),
        pltpu.VMEM((2, _HR, 128), jnp.float32),
        pltpu.VMEM((2, _HR), jnp.int32),
        pltpu.VMEM((2, _HR), jnp.int32),
        pltpu.VMEM((_F, 4, 1, 8, 16), jnp.float32),
        pltpu.SemaphoreType.DMA,
        pltpu.SemaphoreType.DMA,
        pltpu.SemaphoreType.DMA,
        pltpu.SemaphoreType.DMA,
    ],
)
def _gath(xp_hbm, idx_hbm, out5_hbm, idx_v, g_v, q_v, s_v, mo_v,
          gs0, gs1, osem, isem):
    cid = lax.axis_index("c")
    sid = lax.axis_index("s")
    wid = sid * _NC + cid
    gsems = (gs0, gs1)
    iota = lax.iota(jnp.int32, 16)
    zerov = iota * 0

    ipos = pl.multiple_of(wid * 13312, 1024)
    pltpu.async_copy(idx_hbm.at[pl.ds(ipos, 13312)], idx_v, isem).wait()

    def prep(h, b):
        off = h * _HR
        for j in range(_HR // 16):
            x = idx_v[pl.ds(off + j * 16, 16)]
            q_v[b, pl.ds(j * 16, 16)] = lax.shift_right_logical(x, 2)
            s_v[b, pl.ds(j * 16, 16)] = lax.shift_left(x & 3, 5)

    def start_g(b):
        pltpu.async_copy(xp_hbm.at[q_v.at[b]], g_v.at[b], gsems[b])

    def wait_g(b):
        pltpu.make_async_copy(
            xp_hbm.at[pl.ds(0, _HR)], g_v.at[b], gsems[b]).wait()

    def shuffle(b, blbase):
        @plsc.parallel_loop(0, _HR // 16, unroll=2)
        def grp(j):
            iv = j * 16 + iota
            f_v = iv % _F
            bl_v = iv // _F + blbase
            sv = plsc.load_gather(s_v.at[b], [iv])
            for c in range(32):
                tcv = jnp.full((16,), c // 8, jnp.int32)
                c8v = jnp.full((16,), c % 8, jnp.int32)
                v = plsc.load_gather(g_v.at[b], [iv, sv + c])
                plsc.store_scatter(mo_v, [f_v, tcv, zerov, c8v, bl_v], v)

    def start_out(sc):
        b0 = wid * 512 + sc * 16
        tr = b0 // 128
        bo = pl.multiple_of(b0 % 128, 16)
        pltpu.async_copy(
            mo_v, out5_hbm.at[:, :, pl.ds(tr, 1), :, pl.ds(bo, 16)], osem)

    def wait_out():
        pltpu.make_async_copy(
            mo_v, out5_hbm.at[:, :, pl.ds(0, 1), :, pl.ds(0, 16)],
            osem).wait()

    prep(0, 0)
    start_g(0)
    prep(1, 1)
    start_g(1)

    def body(k, carry):
        for b in range(2):
            h = 2 * k + b
            wait_g(b)

            @pl.when((b == 0) & (k >= 1))
            def _():
                wait_out()

            shuffle(b, b * _HB)

            @pl.when(h + 2 < _NH)
            def _():
                prep(h + 2, b)
                start_g(b)

        start_out(k)
        return carry

    lax.fori_loop(0, _NH // 2, body, 0)
    wait_out()


def kernel(indices, table):
    idx = indices.astype(jnp.int32).reshape(_TOT)
    tail = lax.slice(table, (_NBLK_FULL * 128, 0), (_NUM_EMB, _D))
    tail16 = tail.reshape(16, 128)
    xp = _conv(table.T, tail16)
    out5 = _gath(xp, idx)
    return jnp.transpose(out5, (2, 4, 0, 1, 3)).reshape(_B, _F, _D)


# speedup vs baseline: 2.2102x; 1.8827x over previous
import functools

import jax
import jax.numpy as jnp
from jax import lax
from jax.experimental import pallas as pl
from jax.experimental.pallas import tpu as pltpu
from jax.experimental.pallas import tpu_sc as plsc

_NUM_EMB = 1000000
_D = 32
_B = 16384
_F = 26
_TOT = _B * _F
_NQ = _NUM_EMB // 4

_NC = 2
_NS = 16
_NW = _NC * _NS

_mesh = plsc.VectorSubcoreMesh(core_axis_name="c", subcore_axis_name="s")
_tc_params = pltpu.CompilerParams(
    use_tc_tiling_on_sc=True, needs_layout_passes=False)

_NBLK_FULL = _NUM_EMB // 512
_BPT = 62


@functools.partial(
    pl.kernel,
    mesh=_mesh,
    compiler_params=_tc_params,
    out_type=jax.ShapeDtypeStruct((_NQ, 128), jnp.float32),
    scratch_types=[
        pltpu.VMEM((2, 32, 512), jnp.float32),
        pltpu.VMEM((2, 128, 128), jnp.float32),
        pltpu.SemaphoreType.DMA,
        pltpu.SemaphoreType.DMA,
        pltpu.SemaphoreType.DMA,
        pltpu.SemaphoreType.DMA,
    ],
)
def _conv(tT_hbm, tail_hbm, xp_hbm, s_v, o_v, si0, si1, so0, so1):
    wid = lax.axis_index("s") * _NC + lax.axis_index("c")
    base = wid * _BPT
    isems = (si0, si1)
    osems = (so0, so1)
    iota = lax.iota(jnp.int32, 16)
    iota_d4 = iota // 4
    iota_m4_32 = (iota % 4) * 32

    def start_in(g, b):
        gblk = base + g

        @pl.when(gblk < _NBLK_FULL)
        def _():
            r0 = pl.multiple_of(gblk * 512, 512)
            pltpu.async_copy(
                tT_hbm.at[:, pl.ds(r0, 512)], s_v.at[b], isems[b])

    def wait_in(g, b):
        gblk = base + g

        @pl.when(gblk < _NBLK_FULL)
        def _():
            pltpu.make_async_copy(
                tT_hbm.at[:, pl.ds(0, 512)], s_v.at[b], isems[b]).wait()

    def shuffle(g, b):
        gblk = base + g

        @pl.when(gblk < _NBLK_FULL)
        def _():
            for rl0 in range(0, 128, 16):
                rowv = (rl0 // 4) + iota_d4
                for c in range(32):
                    v = s_v[b, c, pl.ds(rl0, 16)]
                    plsc.store_scatter(
                        o_v.at[b], [rowv, iota_m4_32 + c], v)

    def start_out(g, b):
        gblk = base + g

        @pl.when(gblk < _NBLK_FULL)
        def _():
            q0 = pl.multiple_of(gblk * 128, 128)
            pltpu.async_copy(o_v.at[b], xp_hbm.at[pl.ds(q0, 128)], osems[b])

    def wait_out(g, b):
        gblk = base + g

        @pl.when(gblk < _NBLK_FULL)
        def _():
            pltpu.make_async_copy(
                o_v.at[b], xp_hbm.at[pl.ds(0, 128)], osems[b]).wait()

    start_in(0, 0)
    start_in(1, 1)

    def body(k, carry):
        for b in range(2):
            g = 2 * k + b
            wait_in(g, b)

            @pl.when(g >= 2)
            def _():
                wait_out(g - 2, b)

            shuffle(g, b)
            start_out(g, b)

            @pl.when(g + 2 < _BPT)
            def _():
                start_in(g + 2, b)
        return carry

    lax.fori_loop(0, _BPT // 2, body, 0)
    wait_out(_BPT - 2, 0)
    wait_out(_BPT - 1, 1)

    @pl.when(wid == _NW - 1)
    def _():
        pltpu.sync_copy(tail_hbm, o_v.at[0, pl.ds(0, 16)])
        pltpu.sync_copy(o_v.at[0, pl.ds(0, 16)],
                        xp_hbm.at[pl.ds(_NBLK_FULL * 128, 16)])


_RS = _F * 16
_NSC = 32

_sc_params = pltpu.CompilerParams(
    use_tc_tiling_on_sc=False, needs_layout_passes=False)


@functools.partial(
    pl.kernel,
    mesh=_mesh,
    compiler_params=_sc_params,
    out_type=jax.ShapeDtypeStruct((_F, 4, 128, 8, 128), jnp.float32),
    scratch_types=[
        pltpu.VMEM((13312,), jnp.int32),
        pltpu.VMEM((2, _RS, _D), jnp.float32),
        pltpu.VMEM((_F, 4, 1, 8, 16), jnp.float32),
        pltpu.SemaphoreType.DMA,
        pltpu.SemaphoreType.DMA,
        pltpu.SemaphoreType.DMA,
        pltpu.SemaphoreType.DMA,
    ],
)
def _gath(xp_hbm, idx_hbm, out5_hbm, idx_v, g_v, mo_v, gs0, gs1, osem, isem):
    cid = lax.axis_index("c")
    sid = lax.axis_index("s")
    wid = sid * _NC + cid
    gsems = (gs0, gs1)
    iota = lax.iota(jnp.int32, 16)
    zerov = iota * 0

    ipos = pl.multiple_of(wid * 13312, 1024)
    pltpu.async_copy(idx_hbm.at[pl.ds(ipos, 13312)], idx_v, isem).wait()

    def start_g(sc, b):
        off = pl.multiple_of(sc * _RS, 8)
        pltpu.async_copy(
            xp_hbm.at[idx_v.at[pl.ds(off, _RS)]], g_v.at[b], gsems[b])

    def wait_g(b):
        pltpu.make_async_copy(
            xp_hbm.at[pl.ds(0, _RS)], g_v.at[b], gsems[b]).wait()

    def shuffle(b):
        @plsc.parallel_loop(0, _RS // 16, unroll=2)
        def grp(j):
            iv = j * 16 + iota
            f_v = iv % _F
            bl_v = iv // _F
            for c in range(32):
                cv = jnp.full((16,), c, jnp.int32)
                tcv = jnp.full((16,), c // 8, jnp.int32)
                c8v = jnp.full((16,), c % 8, jnp.int32)
                v = plsc.load_gather(g_v.at[b], [iv, cv])
                plsc.store_scatter(mo_v, [f_v, tcv, zerov, c8v, bl_v], v)

    def start_out(sc):
        b0 = wid * 512 + sc * 16
        tr = b0 // 128
        bo = pl.multiple_of(b0 % 128, 16)
        pltpu.async_copy(
            mo_v, out5_hbm.at[:, :, pl.ds(tr, 1), :, pl.ds(bo, 16)], osem)

    def wait_out():
        pltpu.make_async_copy(
            mo_v, out5_hbm.at[:, :, pl.ds(0, 1), :, pl.ds(0, 16)],
            osem).wait()

    start_g(0, 0)
    start_g(1, 1)

    def body(k, carry):
        for b in range(2):
            sc = 2 * k + b
            wait_g(b)

            @pl.when(sc >= 1)
            def _():
                wait_out()

            shuffle(b)
            start_out(sc)

            @pl.when(sc + 2 < _NSC)
            def _():
                start_g(sc + 2, b)
        return carry

    lax.fori_loop(0, _NSC // 2, body, 0)
    wait_out()


def kernel(indices, table):
    idx = indices.astype(jnp.int32).reshape(_TOT)
    tail = lax.slice(table, (_NUM_EMB - 64, 0), (_NUM_EMB, _D))
    tail16 = tail.reshape(16, 128)
    xp = _conv(table.T, tail16)
    out5 = _gath(xp.reshape(_NUM_EMB, _D), idx)
    return jnp.transpose(out5, (2, 4, 0, 1, 3)).reshape(_B, _F, _D)
